# baseline (speedup 1.0000x reference)
import functools

import jax
import jax.numpy as jnp
from jax import lax
from jax.experimental import pallas as pl
from jax.experimental.pallas import tpu as pltpu
from jax.experimental.pallas import tpu_sc as plsc

B, S, NSW = 1024, 50, 20
EMB = 64
ROWW = 128
VPAD = 1000008
T = B * S
NC, NS = 2, 16
NW = NC * NS
TPW = T // NW
IPW = TPW * NSW
CHUNK_TOK = 16
ROWS_PER_CHUNK = CHUNK_TOK * NSW
GATHER_N = 4
GATHER_IDX = ROWS_PER_CHUNK // GATHER_N
NCHUNK = TPW // CHUNK_TOK


def _sc_kernel(table_hbm, idx_hbm, out_hbm,
               idx_v, rows_v0, rows_v1, out_v, sem0, sem1):
    wid = lax.axis_index("s") * NC + lax.axis_index("c")
    out_base = wid * (TPW * EMB)

    pltpu.sync_copy(idx_hbm.at[pl.ds(wid * IPW, IPW)], idx_v)

    def fire(g, rows_v, sem):
        for j in range(GATHER_N):
            o = g * ROWS_PER_CHUNK + j * GATHER_IDX
            pltpu.async_copy(
                table_hbm.at[idx_v.at[pl.ds(o, GATHER_IDX)]],
                rows_v.at[pl.ds(j * GATHER_IDX, GATHER_IDX)],
                sem,
            )

    def drain(rows_v, sem):
        pltpu.make_async_copy(
            table_hbm.at[pl.ds(0, ROWS_PER_CHUNK)], rows_v, sem
        ).wait()

    def compute(g, rows_v):
        @plsc.parallel_loop(0, CHUNK_TOK, unroll=2)
        def _(t):
            r0 = t * NSW
            for c in range(EMB // 16):
                lanes = pl.ds(c * 16, 16)
                acc_a = rows_v[r0, lanes] + rows_v[r0 + 1, lanes]
                acc_b = rows_v[r0 + 2, lanes] + rows_v[r0 + 3, lanes]
                for s in range(4, NSW, 2):
                    acc_a = acc_a + rows_v[r0 + s, lanes]
                    acc_b = acc_b + rows_v[r0 + s + 1, lanes]
                out_v[pl.ds(t * EMB + c * 16, 16)] = (acc_a + acc_b) * (1.0 / NSW)
        pltpu.sync_copy(
            out_v,
            out_hbm.at[pl.ds(out_base + g * (CHUNK_TOK * EMB), CHUNK_TOK * EMB)],
        )

    fire(0, rows_v0, sem0)

    @pl.loop(0, NCHUNK, step=2)
    def _(g):
        fire(g + 1, rows_v1, sem1)
        drain(rows_v0, sem0)
        compute(g, rows_v0)

        @pl.when(g + 2 < NCHUNK)
        def _():
            fire(g + 2, rows_v0, sem0)

        drain(rows_v1, sem1)
        compute(g + 1, rows_v1)


@jax.jit
def _pooled_lookup(table, idx_flat):
    mesh = plsc.VectorSubcoreMesh(core_axis_name="c", subcore_axis_name="s")
    run = pl.kernel(
        _sc_kernel,
        out_type=jax.ShapeDtypeStruct((T * EMB,), jnp.float32),
        mesh=mesh,
        compiler_params=pltpu.CompilerParams(use_tc_tiling_on_sc=True),
        scratch_types=[
            pltpu.VMEM((IPW,), jnp.int32),
            pltpu.VMEM((ROWS_PER_CHUNK, ROWW), jnp.float32),
            pltpu.VMEM((ROWS_PER_CHUNK, ROWW), jnp.float32),
            pltpu.VMEM((CHUNK_TOK * EMB,), jnp.float32),
            pltpu.SemaphoreType.DMA,
            pltpu.SemaphoreType.DMA,
        ],
    )
    tablep = jnp.pad(table, ((0, VPAD - table.shape[0]), (0, ROWW - EMB)))
    return run(tablep, idx_flat)


def kernel(inpt, table):
    idx_flat = inpt.reshape(T * NSW)
    out = _pooled_lookup(table, idx_flat)
    return out.reshape(B, S, EMB)


# speedup vs baseline: 1.0232x; 1.0232x over previous
import functools

import jax
import jax.numpy as jnp
from jax import lax
from jax.experimental import pallas as pl
from jax.experimental.pallas import tpu as pltpu
from jax.experimental.pallas import tpu_sc as plsc

B, S, NSW = 1024, 50, 20
EMB = 64
ROWW = 128
VPAD = 1000008
T = B * S
NC, NS = 2, 16
NW = NC * NS
TPW = T // NW
IPW = TPW * NSW
CHUNK_TOK = 16
ROWS_PER_CHUNK = CHUNK_TOK * NSW
GATHER_N = 4
GATHER_IDX = ROWS_PER_CHUNK // GATHER_N
NCHUNK = TPW // CHUNK_TOK


def _sc_kernel(table_hbm, idx_hbm, out_hbm,
               idx_v, rows_v0, rows_v1, out_v, sem0, sem1):
    wid = lax.axis_index("s") * NC + lax.axis_index("c")
    out_base = wid * (TPW * EMB)

    pltpu.sync_copy(idx_hbm.at[pl.ds(wid * IPW, IPW)], idx_v)

    def fire(g, rows_v, sem):
        for o, n in ((0, 128), (128, 128), (256, 64)):
            pltpu.async_copy(
                table_hbm.at[idx_v.at[pl.ds(g * ROWS_PER_CHUNK + o, n)]],
                rows_v.at[pl.ds(o, n)],
                sem,
            )

    def drain(rows_v, sem):
        pltpu.make_async_copy(
            table_hbm.at[pl.ds(0, ROWS_PER_CHUNK)], rows_v, sem
        ).wait()

    def compute(g, rows_v):
        @pl.loop(0, CHUNK_TOK)
        def _(t):
            r0 = t * NSW
            for c in range(EMB // 16):
                lanes = pl.ds(c * 16, 16)
                acc_a = rows_v[r0, lanes] + rows_v[r0 + 1, lanes]
                acc_b = rows_v[r0 + 2, lanes] + rows_v[r0 + 3, lanes]
                for s in range(4, NSW, 2):
                    acc_a = acc_a + rows_v[r0 + s, lanes]
                    acc_b = acc_b + rows_v[r0 + s + 1, lanes]
                out_v[pl.ds(t * EMB + c * 16, 16)] = (acc_a + acc_b) * (1.0 / NSW)
        pltpu.sync_copy(
            out_v,
            out_hbm.at[pl.ds(out_base + g * (CHUNK_TOK * EMB), CHUNK_TOK * EMB)],
        )

    fire(0, rows_v0, sem0)

    @pl.loop(0, NCHUNK, step=2)
    def _(g):
        fire(g + 1, rows_v1, sem1)
        drain(rows_v0, sem0)
        compute(g, rows_v0)

        @pl.when(g + 2 < NCHUNK)
        def _():
            fire(g + 2, rows_v0, sem0)

        drain(rows_v1, sem1)
        compute(g + 1, rows_v1)


@jax.jit
def _pooled_lookup(table, idx_flat):
    mesh = plsc.VectorSubcoreMesh(core_axis_name="c", subcore_axis_name="s")
    run = pl.kernel(
        _sc_kernel,
        out_type=jax.ShapeDtypeStruct((T * EMB,), jnp.float32),
        mesh=mesh,
        compiler_params=pltpu.CompilerParams(use_tc_tiling_on_sc=True),
        scratch_types=[
            pltpu.VMEM((IPW,), jnp.int32),
            pltpu.VMEM((ROWS_PER_CHUNK, ROWW), jnp.float32),
            pltpu.VMEM((ROWS_PER_CHUNK, ROWW), jnp.float32),
            pltpu.VMEM((CHUNK_TOK * EMB,), jnp.float32),
            pltpu.SemaphoreType.DMA,
            pltpu.SemaphoreType.DMA,
        ],
    )
    tablep = jnp.pad(table, ((0, VPAD - table.shape[0]), (0, ROWW - EMB)))
    return run(tablep, idx_flat)


def kernel(inpt, table):
    idx_flat = inpt.reshape(T * NSW)
    out = _pooled_lookup(table, idx_flat)
    return out.reshape(B, S, EMB)
